# baseline (speedup 1.0000x reference)
import functools

import jax
import jax.numpy as jnp
from jax.experimental import pallas as pl


def _silu(v):
    return v * jax.lax.logistic(v)


def _layer_body(h_ref, hi_ref, x_ref, xi_ref, xT_ref, xjrow_ref, kt_ref, k_ref,
                we1a_ref, we1b_ref, we1c_ref, be1_ref,
                we2_ref, be2_ref, wx1_ref, bx1_ref, wx2r_ref, bx2_ref,
                wh1a_ref, wh1b_ref, bh1_ref, wh2_ref, bh2_ref,
                oh_ref, ox_ref, *, bi, n, d):
    i = pl.program_id(0)
    r0 = i * bi
    f32 = jnp.float32

    h_all = h_ref[:, :]
    hi = hi_ref[:, :]
    xi = xi_ref[:, :]
    we1c = we1c_ref[0, :][None, :]

    cross = jnp.dot(xi, xT_ref[:, :], preferred_element_type=f32)
    xisq = jnp.sum(xi * xi, axis=1, keepdims=True)
    xjsq = jnp.sum(x_ref[:, :] * x_ref[:, :], axis=1, keepdims=True)

    ai = jnp.dot(hi, we1a_ref[:, :], preferred_element_type=f32)
    bj = jnp.dot(h_all, we1b_ref[:, :], preferred_element_type=f32)
    aip = ai + xisq * we1c + be1_ref[0, :][None, :]
    bjp = bj + xjsq * we1c
    m0 = (aip[:, None, :] + bjp[None, :, :]
          + (-2.0 * cross)[:, :, None] * we1c[None, :, :])
    m1 = _silu(m0).reshape(bi * n, d)
    m = _silu(jnp.dot(m1, we2_ref[:, :], preferred_element_type=f32)
              + be2_ref[0, :][None, :])

    t = _silu(jnp.dot(m, wx1_ref[:, :], preferred_element_type=f32)
              + bx1_ref[0, :][None, :])
    c_col = jnp.dot(t, wx2r_ref[:, :].T, preferred_element_type=f32)
    c_row = c_col.reshape(1, bi * n)

    v = jnp.concatenate([c_row * xjrow_ref[:, :], c_row], axis=0)
    cv = jnp.dot(v, k_ref[:, :], preferred_element_type=f32).T
    bx2 = bx2_ref[0, 0]
    xsum = jnp.sum(x_ref[:, :], axis=0, keepdims=True)
    cxu = cv[:, 0:3] + bx2 * xsum
    csumu = cv[:, 3:4] + n * bx2

    maggu = jnp.dot(kt_ref[:, :], m, preferred_element_type=f32)

    bj_diag = jnp.dot(hi, we1b_ref[:, :], preferred_element_type=f32)
    m0d = ai + bj_diag + be1_ref[0, :][None, :]
    md = _silu(jnp.dot(_silu(m0d), we2_ref[:, :], preferred_element_type=f32)
               + be2_ref[0, :][None, :])
    td = _silu(jnp.dot(md, wx1_ref[:, :], preferred_element_type=f32)
               + bx1_ref[0, :][None, :])
    cd = jnp.sum(td * wx2r_ref[0, :][None, :], axis=1, keepdims=True) + bx2

    csum = csumu - cd
    cx = cxu - cd * xi
    ox_ref[:, :] = xi + (xi * csum - cx) * (1.0 / (n - 1))

    magg = maggu - md
    g = _silu(jnp.dot(hi, wh1a_ref[:, :], preferred_element_type=f32)
              + jnp.dot(magg, wh1b_ref[:, :], preferred_element_type=f32)
              + bh1_ref[0, :][None, :])
    hupd = jnp.dot(g, wh2_ref[:, :], preferred_element_type=f32) \
        + bh2_ref[0, :][None, :]
    oh_ref[:, :] = _silu(hi + hupd)


def _egnn_layer(h, x, We1, be1, We2, be2, Wx1, bx1, Wx2, bx2,
                Wh1, bh1, Wh2, bh2, *, bi, interpret=False):
    n, d = h.shape
    xT = x.T
    xjrow = jnp.tile(xT, (1, bi))
    kt = jnp.repeat(jnp.eye(bi, dtype=jnp.float32), n, axis=1)
    kk = kt.T
    we1a = We1[:d]
    we1b = We1[d:2 * d]
    we1c = We1[2 * d:2 * d + 1]
    wx2r = Wx2.T
    bx2m = bx2.reshape(1, 1)
    wh1a = Wh1[:d]
    wh1b = Wh1[d:]

    full = lambda shape: pl.BlockSpec(shape, lambda i: (0, 0))
    body = functools.partial(_layer_body, bi=bi, n=n, d=d)
    return pl.pallas_call(
        body,
        grid=(n // bi,),
        in_specs=[
            full((n, d)),
            pl.BlockSpec((bi, d), lambda i: (i, 0)),
            full((n, 3)),
            pl.BlockSpec((bi, 3), lambda i: (i, 0)),
            full((3, n)),
            full((3, bi * n)),
            full((bi, bi * n)),
            full((bi * n, bi)),
            full((d, d)),
            full((d, d)),
            full((1, d)),
            full((1, d)),
            full((d, d)),
            full((1, d)),
            full((d, d)),
            full((1, d)),
            full((1, d)),
            full((1, 1)),
            full((d, d)),
            full((d, d)),
            full((1, d)),
            full((d, d)),
            full((1, d)),
        ],
        out_specs=[
            pl.BlockSpec((bi, d), lambda i: (i, 0)),
            pl.BlockSpec((bi, 3), lambda i: (i, 0)),
        ],
        out_shape=[
            jax.ShapeDtypeStruct((n, d), jnp.float32),
            jax.ShapeDtypeStruct((n, 3), jnp.float32),
        ],
        interpret=interpret,
    )(h, h, x, x, xT, xjrow, kt, kk, we1a, we1b, we1c, be1.reshape(1, d),
      We2, be2.reshape(1, d), Wx1, bx1.reshape(1, d), wx2r, bx2m,
      wh1a, wh1b, bh1.reshape(1, d), Wh2, bh2.reshape(1, d))


def kernel(h, x, We1, be1, We2, be2, Wx1, bx1, Wx2, bx2, Wh1, bh1, Wh2, bh2):
    L = We1.shape[0]
    for l in range(L):
        h, x = _egnn_layer(h, x, We1[l], be1[l], We2[l], be2[l],
                           Wx1[l], bx1[l], Wx2[l], bx2[l],
                           Wh1[l], bh1[l], Wh2[l], bh2[l], bi=16)
    return (h, x)


# speedup vs baseline: 2.0795x; 2.0795x over previous
import functools

import jax
import jax.numpy as jnp
from jax.experimental import pallas as pl


def _silu(v):
    return v * jax.lax.logistic(v)


def _layer_body(h_ref, hi_ref, x_ref, xi_ref, xT_ref, xjf_ref,
                we1a_ref, we1b_ref, we1c_ref, be1_ref,
                we2_ref, be2_ref, wx1_ref, bx1_ref, wx2r_ref, bx2_ref,
                wh1a_ref, wh1b_ref, bh1_ref, wh2_ref, bh2_ref,
                oh_ref, ox_ref, *, bi, n, d):
    i = pl.program_id(0)
    r0 = i * bi
    f32 = jnp.float32

    h_all = h_ref[:, :]
    hi = hi_ref[:, :]
    xi = xi_ref[:, :]
    we1c = we1c_ref[0, :][None, :]

    cross = jnp.dot(xi, xT_ref[:, :], preferred_element_type=f32)
    xisq = jnp.sum(xi * xi, axis=1, keepdims=True)
    xjsq = jnp.sum(x_ref[:, :] * x_ref[:, :], axis=1, keepdims=True)

    ai = jnp.dot(hi, we1a_ref[:, :], preferred_element_type=f32)
    bj = jnp.dot(h_all, we1b_ref[:, :], preferred_element_type=f32)
    aip = ai + xisq * we1c + be1_ref[0, :][None, :]
    bjp = bj + xjsq * we1c
    m0 = (aip[:, None, :] + bjp[None, :, :]
          + (-2.0 * cross)[:, :, None] * we1c[None, :, :])
    m1 = _silu(m0).reshape(bi * n, d)
    m = _silu(jnp.dot(m1, we2_ref[:, :], preferred_element_type=f32)
              + be2_ref[0, :][None, :])

    t = _silu(jnp.dot(m, wx1_ref[:, :], preferred_element_type=f32)
              + bx1_ref[0, :][None, :])
    c_col = jnp.dot(t, wx2r_ref[:, :].T, preferred_element_type=f32)

    u = c_col * xjf_ref[:, :]
    cv = jnp.sum(u.reshape(bi, n, 4), axis=1)
    bx2 = bx2_ref[0, 0]
    xsum = jnp.sum(x_ref[:, :], axis=0, keepdims=True)
    cxu = cv[:, 0:3] + bx2 * xsum
    csumu = cv[:, 3:4] + n * bx2

    maggu = jnp.sum(m.reshape(bi, n, d), axis=1)

    bj_diag = jnp.dot(hi, we1b_ref[:, :], preferred_element_type=f32)
    m0d = ai + bj_diag + be1_ref[0, :][None, :]
    md = _silu(jnp.dot(_silu(m0d), we2_ref[:, :], preferred_element_type=f32)
               + be2_ref[0, :][None, :])
    td = _silu(jnp.dot(md, wx1_ref[:, :], preferred_element_type=f32)
               + bx1_ref[0, :][None, :])
    cd = jnp.sum(td * wx2r_ref[0, :][None, :], axis=1, keepdims=True) + bx2

    csum = csumu - cd
    cx = cxu - cd * xi
    ox_ref[:, :] = xi + (xi * csum - cx) * (1.0 / (n - 1))

    magg = maggu - md
    g = _silu(jnp.dot(hi, wh1a_ref[:, :], preferred_element_type=f32)
              + jnp.dot(magg, wh1b_ref[:, :], preferred_element_type=f32)
              + bh1_ref[0, :][None, :])
    hupd = jnp.dot(g, wh2_ref[:, :], preferred_element_type=f32) \
        + bh2_ref[0, :][None, :]
    oh_ref[:, :] = _silu(hi + hupd)


def _egnn_layer(h, x, We1, be1, We2, be2, Wx1, bx1, Wx2, bx2,
                Wh1, bh1, Wh2, bh2, *, bi, interpret=False):
    n, d = h.shape
    xT = x.T
    xjf = jnp.tile(jnp.concatenate([x, jnp.ones((n, 1), jnp.float32)], axis=1),
                   (bi, 1))
    we1a = We1[:d]
    we1b = We1[d:2 * d]
    we1c = We1[2 * d:2 * d + 1]
    wx2r = Wx2.T
    bx2m = bx2.reshape(1, 1)
    wh1a = Wh1[:d]
    wh1b = Wh1[d:]

    full = lambda shape: pl.BlockSpec(shape, lambda i: (0, 0))
    body = functools.partial(_layer_body, bi=bi, n=n, d=d)
    return pl.pallas_call(
        body,
        grid=(n // bi,),
        in_specs=[
            full((n, d)),
            pl.BlockSpec((bi, d), lambda i: (i, 0)),
            full((n, 3)),
            pl.BlockSpec((bi, 3), lambda i: (i, 0)),
            full((3, n)),
            full((bi * n, 4)),
            full((d, d)),
            full((d, d)),
            full((1, d)),
            full((1, d)),
            full((d, d)),
            full((1, d)),
            full((d, d)),
            full((1, d)),
            full((1, d)),
            full((1, 1)),
            full((d, d)),
            full((d, d)),
            full((1, d)),
            full((d, d)),
            full((1, d)),
        ],
        out_specs=[
            pl.BlockSpec((bi, d), lambda i: (i, 0)),
            pl.BlockSpec((bi, 3), lambda i: (i, 0)),
        ],
        out_shape=[
            jax.ShapeDtypeStruct((n, d), jnp.float32),
            jax.ShapeDtypeStruct((n, 3), jnp.float32),
        ],
        interpret=interpret,
    )(h, h, x, x, xT, xjf, we1a, we1b, we1c, be1.reshape(1, d),
      We2, be2.reshape(1, d), Wx1, bx1.reshape(1, d), wx2r, bx2m,
      wh1a, wh1b, bh1.reshape(1, d), Wh2, bh2.reshape(1, d))


def kernel(h, x, We1, be1, We2, be2, Wx1, bx1, Wx2, bx2, Wh1, bh1, Wh2, bh2):
    L = We1.shape[0]
    for l in range(L):
        h, x = _egnn_layer(h, x, We1[l], be1[l], We2[l], be2[l],
                           Wx1[l], bx1[l], Wx2[l], bx2[l],
                           Wh1[l], bh1[l], Wh2[l], bh2[l], bi=16)
    return (h, x)
